# baseline (speedup 1.0000x reference)
import functools

import jax
import jax.numpy as jnp
from jax.experimental import pallas as pl
from jax.experimental.pallas import tpu as pltpu

_BLOCK_ROWS = 10000


def _fused_kernel(p_ref, q_ref, ic_ref, w_ref, weu_ref, wei_ref,
                  user_out_ref, item_out_ref):
    f32 = jnp.float32
    user_out_ref[...] = jnp.dot(p_ref[...], weu_ref[...],
                                preferred_element_type=f32)
    wei_top = wei_ref[0:64, :]
    wei_bot = wei_ref[64:128, :]
    w_fold = jnp.dot(w_ref[...], wei_bot, preferred_element_type=f32)
    item_out_ref[...] = (
        jnp.dot(q_ref[...], wei_top, preferred_element_type=f32)
        + jnp.dot(ic_ref[...], w_fold, preferred_element_type=f32)
    )


@jax.jit
def kernel(P, Q, item_content, W, weu, wei):
    n = P.shape[0]
    d = weu.shape[1]
    grid = (n // _BLOCK_ROWS,)
    row_block = lambda i: (i, 0)
    const_block = lambda i: (0, 0)
    user_emb, item_emb = pl.pallas_call(
        _fused_kernel,
        grid=grid,
        in_specs=[
            pl.BlockSpec((_BLOCK_ROWS, P.shape[1]), row_block),
            pl.BlockSpec((_BLOCK_ROWS, Q.shape[1]), row_block),
            pl.BlockSpec((_BLOCK_ROWS, item_content.shape[1]), row_block),
            pl.BlockSpec(W.shape, const_block),
            pl.BlockSpec(weu.shape, const_block),
            pl.BlockSpec(wei.shape, const_block),
        ],
        out_specs=[
            pl.BlockSpec((_BLOCK_ROWS, d), row_block),
            pl.BlockSpec((_BLOCK_ROWS, d), row_block),
        ],
        out_shape=[
            jax.ShapeDtypeStruct((n, d), jnp.float32),
            jax.ShapeDtypeStruct((n, d), jnp.float32),
        ],
        compiler_params=pltpu.CompilerParams(
            dimension_semantics=("parallel",),
            flags={"XLA_SET_SPLIT_INPUT_OUTPUT_DMAS": True},
        ),
    )(P, Q, item_content, W, weu, wei)
    return (user_emb, item_emb)


# speedup vs baseline: 1.0374x; 1.0374x over previous
import jax
import jax.numpy as jnp
from jax.experimental import pallas as pl
from jax.experimental.pallas import tpu as pltpu

_BN = 5000
_NBUF = 3


def _make_body(rows_per_core, nchunk):
    def body(p_hbm, q_hbm, ic_hbm, w_hbm, weu_hbm, wei_hbm,
             uo_hbm, io_hbm,
             p_buf, q_buf, ic_buf, uo_buf, io_buf,
             w_vmem, weu_vmem, wei_vmem,
             in_sems, out_sems, w_sems):
        f32 = jnp.float32
        core = jax.lax.axis_index("core")
        base = core * rows_per_core

        wc = (
            pltpu.make_async_copy(w_hbm, w_vmem, w_sems.at[0]),
            pltpu.make_async_copy(weu_hbm, weu_vmem, w_sems.at[1]),
            pltpu.make_async_copy(wei_hbm, wei_vmem, w_sems.at[2]),
        )
        for c in wc:
            c.start()

        def in_copies(slot, i):
            r = pl.ds(base + i * _BN, _BN)
            return (
                pltpu.make_async_copy(p_hbm.at[r, :], p_buf.at[slot],
                                      in_sems.at[slot, 0]),
                pltpu.make_async_copy(q_hbm.at[r, :], q_buf.at[slot],
                                      in_sems.at[slot, 1]),
                pltpu.make_async_copy(ic_hbm.at[r, :], ic_buf.at[slot],
                                      in_sems.at[slot, 2]),
            )

        def out_copies(slot, i):
            r = pl.ds(base + i * _BN, _BN)
            return (
                pltpu.make_async_copy(uo_buf.at[slot], uo_hbm.at[r, :],
                                      out_sems.at[slot, 0]),
                pltpu.make_async_copy(io_buf.at[slot], io_hbm.at[r, :],
                                      out_sems.at[slot, 1]),
            )

        for k in range(min(_NBUF, nchunk)):
            for c in in_copies(k, k):
                c.start()

        for c in wc:
            c.wait()
        weu_v = weu_vmem[...]
        wei_top = wei_vmem[0:64, :]
        w_fold = jnp.dot(w_vmem[...], wei_vmem[64:128, :],
                         preferred_element_type=f32)

        for i in range(nchunk):
            slot = i % _NBUF
            for c in in_copies(slot, i):
                c.wait()
            if i >= _NBUF:
                for c in out_copies(slot, i - _NBUF):
                    c.wait()
            uo_buf[slot] = jnp.dot(p_buf[slot], weu_v,
                                   preferred_element_type=f32)
            io_buf[slot] = (
                jnp.dot(q_buf[slot], wei_top, preferred_element_type=f32)
                + jnp.dot(ic_buf[slot], w_fold, preferred_element_type=f32)
            )
            for c in out_copies(slot, i):
                c.start()
            nxt = i + _NBUF
            if nxt < nchunk:
                for c in in_copies(slot, nxt):
                    c.start()

        for i in range(max(nchunk - _NBUF, 0), nchunk):
            for c in out_copies(i % _NBUF, i):
                c.wait()

    return body


@jax.jit
def kernel(P, Q, item_content, W, weu, wei):
    n = P.shape[0]
    d = weu.shape[1]
    f32 = jnp.float32
    mesh = pltpu.create_tensorcore_mesh("core")
    ncores = mesh.shape["core"]
    rows_per_core = n // ncores
    nchunk = rows_per_core // _BN

    run = pl.kernel(
        _make_body(rows_per_core, nchunk),
        out_type=[
            jax.ShapeDtypeStruct((n, d), f32),
            jax.ShapeDtypeStruct((n, d), f32),
        ],
        mesh=mesh,
        scratch_types=[
            pltpu.VMEM((_NBUF, _BN, P.shape[1]), f32),
            pltpu.VMEM((_NBUF, _BN, Q.shape[1]), f32),
            pltpu.VMEM((_NBUF, _BN, item_content.shape[1]), f32),
            pltpu.VMEM((_NBUF, _BN, d), f32),
            pltpu.VMEM((_NBUF, _BN, d), f32),
            pltpu.VMEM(W.shape, f32),
            pltpu.VMEM(weu.shape, f32),
            pltpu.VMEM(wei.shape, f32),
            pltpu.SemaphoreType.DMA((_NBUF, 3)),
            pltpu.SemaphoreType.DMA((_NBUF, 2)),
            pltpu.SemaphoreType.DMA((3,)),
        ],
    )
    user_emb, item_emb = run(P, Q, item_content, W, weu, wei)
    return (user_emb, item_emb)


# speedup vs baseline: 1.0859x; 1.0467x over previous
import jax
import jax.numpy as jnp
from jax.experimental import pallas as pl
from jax.experimental.pallas import tpu as pltpu

_BN_USER = 25000
_BN_ITEM = 10000


def _user_kernel(p_ref, weu_ref, out_ref):
    out_ref[...] = jnp.dot(p_ref[...], weu_ref[...],
                           preferred_element_type=jnp.float32)


def _item_kernel(q_ref, ic_ref, w_ref, wei_ref, out_ref):
    f32 = jnp.float32
    wei_top = wei_ref[0:64, :]
    w_fold = jnp.dot(w_ref[...], wei_ref[64:128, :], preferred_element_type=f32)
    out_ref[...] = (
        jnp.dot(q_ref[...], wei_top, preferred_element_type=f32)
        + jnp.dot(ic_ref[...], w_fold, preferred_element_type=f32)
    )


@jax.jit
def kernel(P, Q, item_content, W, weu, wei):
    n = P.shape[0]
    d = weu.shape[1]
    f32 = jnp.float32
    row = lambda i: (i, 0)
    const = lambda i: (0, 0)

    user_emb = pl.pallas_call(
        _user_kernel,
        grid=(n // _BN_USER,),
        in_specs=[
            pl.BlockSpec((_BN_USER, P.shape[1]), row),
            pl.BlockSpec(weu.shape, const),
        ],
        out_specs=pl.BlockSpec((_BN_USER, d), row),
        out_shape=jax.ShapeDtypeStruct((n, d), f32),
        compiler_params=pltpu.CompilerParams(
            dimension_semantics=("parallel",),
        ),
    )(P, weu)

    item_emb = pl.pallas_call(
        _item_kernel,
        grid=(n // _BN_ITEM,),
        in_specs=[
            pl.BlockSpec((_BN_ITEM, Q.shape[1]), row),
            pl.BlockSpec((_BN_ITEM, item_content.shape[1]), row),
            pl.BlockSpec(W.shape, const),
            pl.BlockSpec(wei.shape, const),
        ],
        out_specs=pl.BlockSpec((_BN_ITEM, d), row),
        out_shape=jax.ShapeDtypeStruct((n, d), f32),
        compiler_params=pltpu.CompilerParams(
            dimension_semantics=("parallel",),
        ),
    )(Q, item_content, W, wei)

    return (user_emb, item_emb)


# speedup vs baseline: 1.0920x; 1.0057x over previous
import jax
import jax.numpy as jnp
from jax.experimental import pallas as pl
from jax.experimental.pallas import tpu as pltpu

_BLOCK_ROWS = 10000


def _fused_kernel(p_ref, q_ref, ic_ref, wcat_ref, user_out_ref, item_out_ref):
    f32 = jnp.float32
    w = wcat_ref[0:128, :]
    weu = wcat_ref[128:256, :]
    wei_top = wcat_ref[256:320, :]
    wei_bot = wcat_ref[320:384, :]
    user_out_ref[...] = jnp.dot(p_ref[...], weu, preferred_element_type=f32)
    w_fold = jnp.dot(w, wei_bot, preferred_element_type=f32)
    item_out_ref[...] = (
        jnp.dot(q_ref[...], wei_top, preferred_element_type=f32)
        + jnp.dot(ic_ref[...], w_fold, preferred_element_type=f32)
    )


@jax.jit
def kernel(P, Q, item_content, W, weu, wei):
    n = P.shape[0]
    d = weu.shape[1]
    wcat = jnp.concatenate([W, weu, wei], axis=0)
    grid = (n // _BLOCK_ROWS,)
    row_block = lambda i: (i, 0)
    const_block = lambda i: (0, 0)
    user_emb, item_emb = pl.pallas_call(
        _fused_kernel,
        grid=grid,
        in_specs=[
            pl.BlockSpec((_BLOCK_ROWS, P.shape[1]), row_block),
            pl.BlockSpec((_BLOCK_ROWS, Q.shape[1]), row_block),
            pl.BlockSpec((_BLOCK_ROWS, item_content.shape[1]), row_block),
            pl.BlockSpec(wcat.shape, const_block),
        ],
        out_specs=[
            pl.BlockSpec((_BLOCK_ROWS, d), row_block),
            pl.BlockSpec((_BLOCK_ROWS, d), row_block),
        ],
        out_shape=[
            jax.ShapeDtypeStruct((n, d), jnp.float32),
            jax.ShapeDtypeStruct((n, d), jnp.float32),
        ],
        compiler_params=pltpu.CompilerParams(
            dimension_semantics=("parallel",),
        ),
    )(P, Q, item_content, wcat)
    return (user_emb, item_emb)


# speedup vs baseline: 1.2302x; 1.1265x over previous
import jax
import jax.numpy as jnp
from jax.experimental import pallas as pl
from jax.experimental.pallas import tpu as pltpu

_MAJ = 12500
_BMAJ = 1250


def _fused_kernel(p_ref, q_ref, ic_ref, wcat_ref, user_out_ref, item_out_ref):
    f32 = jnp.float32
    w = wcat_ref[0:128, :]
    weu = wcat_ref[128:256, :]
    wei_top = wcat_ref[256:320, :]
    wei_bot = wcat_ref[320:384, :]
    p = p_ref[...].reshape(_BMAJ * 8, 128)
    q = q_ref[...].reshape(_BMAJ * 8, 64)
    ic = ic_ref[...].reshape(_BMAJ * 8, 128)
    user = jnp.dot(p, weu, preferred_element_type=f32)
    w_fold = jnp.dot(w, wei_bot, preferred_element_type=f32)
    item = (jnp.dot(q, wei_top, preferred_element_type=f32)
            + jnp.dot(ic, w_fold, preferred_element_type=f32))
    user_out_ref[...] = user.reshape(_BMAJ, 8, 64)
    item_out_ref[...] = item.reshape(_BMAJ, 8, 64)


@jax.jit
def kernel(P, Q, item_content, W, weu, wei):
    n = P.shape[0]
    d = weu.shape[1]
    wcat = jnp.concatenate([W, weu, wei], axis=0)
    P3 = P.reshape(_MAJ, 8, P.shape[1])
    Q3 = Q.reshape(_MAJ, 8, Q.shape[1])
    ic3 = item_content.reshape(_MAJ, 8, item_content.shape[1])
    grid = (_MAJ // _BMAJ,)
    row_block = lambda i: (i, 0, 0)
    user3, item3 = pl.pallas_call(
        _fused_kernel,
        grid=grid,
        in_specs=[
            pl.BlockSpec((_BMAJ, 8, P.shape[1]), row_block),
            pl.BlockSpec((_BMAJ, 8, Q.shape[1]), row_block),
            pl.BlockSpec((_BMAJ, 8, item_content.shape[1]), row_block),
            pl.BlockSpec(wcat.shape, lambda i: (0, 0)),
        ],
        out_specs=[
            pl.BlockSpec((_BMAJ, 8, d), row_block),
            pl.BlockSpec((_BMAJ, 8, d), row_block),
        ],
        out_shape=[
            jax.ShapeDtypeStruct((_MAJ, 8, d), jnp.float32),
            jax.ShapeDtypeStruct((_MAJ, 8, d), jnp.float32),
        ],
        compiler_params=pltpu.CompilerParams(
            dimension_semantics=("parallel",),
        ),
    )(P3, Q3, ic3, wcat)
    return (user3.reshape(n, d), item3.reshape(n, d))


# speedup vs baseline: 1.2314x; 1.0010x over previous
import jax
import jax.numpy as jnp
from jax.experimental import pallas as pl
from jax.experimental.pallas import tpu as pltpu

_MAJ = 100
_BMAJ = 10


def _fused_kernel(p_ref, q_ref, ic_ref, wcat_ref, user_out_ref, item_out_ref):
    f32 = jnp.float32
    w = wcat_ref[0:128, :]
    weu = wcat_ref[128:256, :]
    wei_top = wcat_ref[256:320, :]
    wei_bot = wcat_ref[320:384, :]
    p = p_ref[...].reshape(_BMAJ * 1000, 128)
    q = q_ref[...].reshape(_BMAJ * 1000, 64)
    ic = ic_ref[...].reshape(_BMAJ * 1000, 128)
    user = jnp.dot(p, weu, preferred_element_type=f32)
    w_fold = jnp.dot(w, wei_bot, preferred_element_type=f32)
    item = (jnp.dot(q, wei_top, preferred_element_type=f32)
            + jnp.dot(ic, w_fold, preferred_element_type=f32))
    user_out_ref[...] = user.reshape(_BMAJ, 1000, 64)
    item_out_ref[...] = item.reshape(_BMAJ, 1000, 64)


@jax.jit
def kernel(P, Q, item_content, W, weu, wei):
    n = P.shape[0]
    d = weu.shape[1]
    wcat = jnp.concatenate([W, weu, wei], axis=0)
    P3 = P.reshape(_MAJ, 1000, P.shape[1])
    Q3 = Q.reshape(_MAJ, 1000, Q.shape[1])
    ic3 = item_content.reshape(_MAJ, 1000, item_content.shape[1])
    grid = (_MAJ // _BMAJ,)
    row_block = lambda i: (i, 0, 0)
    user3, item3 = pl.pallas_call(
        _fused_kernel,
        grid=grid,
        in_specs=[
            pl.BlockSpec((_BMAJ, 1000, P.shape[1]), row_block),
            pl.BlockSpec((_BMAJ, 1000, Q.shape[1]), row_block),
            pl.BlockSpec((_BMAJ, 1000, item_content.shape[1]), row_block),
            pl.BlockSpec(wcat.shape, lambda i: (0, 0)),
        ],
        out_specs=[
            pl.BlockSpec((_BMAJ, 1000, d), row_block),
            pl.BlockSpec((_BMAJ, 1000, d), row_block),
        ],
        out_shape=[
            jax.ShapeDtypeStruct((_MAJ, 1000, d), jnp.float32),
            jax.ShapeDtypeStruct((_MAJ, 1000, d), jnp.float32),
        ],
        compiler_params=pltpu.CompilerParams(
            dimension_semantics=("parallel",),
        ),
    )(P3, Q3, ic3, wcat)
    return (user3.reshape(n, d), item3.reshape(n, d))
